# Initial kernel scaffold; baseline (speedup 1.0000x reference)
import functools

import jax
import jax.numpy as jnp
from jax import lax
from jax.experimental import pallas as pl
from jax.experimental.pallas import tpu as pltpu
from jax.experimental.pallas import tpu_sc as plsc

NC = 2
NS = 16
NW = NC * NS
CHUNK = 80


def kernel(x, edge_index, edge_pairs, W1, b1, W2, b2):
    f32 = jnp.float32
    N, DIN = x.shape
    DH = W1.shape[1]
    DO = W2.shape[1]
    E = edge_index.shape[1]

    EPT = E // NW
    NCH = EPT // CHUNK
    STR = N // NS

    src = edge_index[0].astype(jnp.int32)
    dst = edge_index[1].astype(jnp.int32)
    uu = edge_pairs[0].astype(jnp.int32)
    vv = edge_pairs[1].astype(jnp.int32)

    zeros16 = jnp.zeros((N, 16), f32)
    zeros32 = jnp.zeros((N, DH), f32)
    e0 = jnp.zeros((CHUNK, 16), f32).at[:, 0].set(1.0)

    mesh = plsc.VectorSubcoreMesh(core_axis_name="c", subcore_axis_name="s")


    @functools.partial(
        pl.kernel,
        out_type=jax.ShapeDtypeStruct((NC, N, 16), f32),
        mesh=mesh,
        scratch_types=[
            pltpu.VMEM((CHUNK,), jnp.int32),
            pltpu.VMEM((CHUNK, 16), f32),
            pltpu.VMEM_SHARED((N, 16), f32),
        ],
    )
    def sc_deg(dst_h, z_h, e0_h, out_h, idx_v, e0_v, acc_sh):
        cid = lax.axis_index("c")
        sid = lax.axis_index("s")
        base = (cid * NS + sid) * EPT
        sp = sid * STR
        pltpu.sync_copy(z_h.at[pl.ds(sp, STR)], acc_sh.at[pl.ds(sp, STR)])
        pltpu.sync_copy(e0_h, e0_v)
        plsc.subcore_barrier()

        @pl.loop(0, NCH)
        def _(k):
            pltpu.sync_copy(dst_h.at[pl.ds(base + k * CHUNK, CHUNK)], idx_v)
            pltpu.sync_copy(e0_v, acc_sh.at[idx_v], add=True)

        plsc.subcore_barrier()
        pltpu.sync_copy(acc_sh.at[pl.ds(sp, STR)], out_h.at[cid, pl.ds(sp, STR)])

    @functools.partial(
        pl.kernel,
        out_type=jax.ShapeDtypeStruct((NC, N, DH), f32),
        mesh=mesh,
        scratch_types=[
            pltpu.VMEM((CHUNK,), jnp.int32),
            pltpu.VMEM((CHUNK,), jnp.int32),
            pltpu.VMEM((CHUNK, DH), f32),
            pltpu.VMEM_SHARED((N, DH), f32),
            pltpu.SemaphoreType.DMA,
        ],
    )
    def sc_seg(src_h, dst_h, tab_h, z_h, out_h, idx_s, idx_d, rows_v, acc_sh, sem):
        cid = lax.axis_index("c")
        sid = lax.axis_index("s")
        base = (cid * NS + sid) * EPT
        sp = sid * STR
        pltpu.sync_copy(z_h.at[pl.ds(sp, STR)], acc_sh.at[pl.ds(sp, STR)])
        plsc.subcore_barrier()

        @pl.loop(0, NCH)
        def _(k):
            off = base + k * CHUNK
            pltpu.sync_copy(src_h.at[pl.ds(off, CHUNK)], idx_s)
            pltpu.async_copy(tab_h.at[idx_s], rows_v, sem).wait()
            pltpu.sync_copy(dst_h.at[pl.ds(off, CHUNK)], idx_d)
            pltpu.sync_copy(rows_v, acc_sh.at[idx_d], add=True)

        plsc.subcore_barrier()
        pltpu.sync_copy(acc_sh.at[pl.ds(sp, STR)], out_h.at[cid, pl.ds(sp, STR)])

    @functools.partial(
        pl.kernel,
        out_type=jax.ShapeDtypeStruct((E, 16), f32),
        mesh=mesh,
        scratch_types=[
            pltpu.VMEM((CHUNK,), jnp.int32),
            pltpu.VMEM((CHUNK,), jnp.int32),
            pltpu.VMEM((CHUNK, DO), f32),
            pltpu.VMEM((CHUNK, DO), f32),
            pltpu.VMEM((CHUNK, 16), f32),
            pltpu.SemaphoreType.DMA,
        ],
    )
    def sc_dec(u_h, v_h, z_h, out_h, iu_v, iv_v, zu_v, zv_v, p_v, sem):
        cid = lax.axis_index("c")
        sid = lax.axis_index("s")
        base = (cid * NS + sid) * EPT

        @pl.loop(0, NCH)
        def _(k):
            off = base + k * CHUNK
            pltpu.sync_copy(u_h.at[pl.ds(off, CHUNK)], iu_v)
            pltpu.sync_copy(v_h.at[pl.ds(off, CHUNK)], iv_v)
            cp1 = pltpu.async_copy(z_h.at[iu_v], zu_v, sem)
            cp2 = pltpu.async_copy(z_h.at[iv_v], zv_v, sem)
            cp1.wait()
            cp2.wait()

            @pl.loop(0, CHUNK)
            def _(r):
                a0 = zu_v[r, pl.ds(0, 16)]
                a1 = zu_v[r, pl.ds(16, 16)]
                b0 = zv_v[r, pl.ds(0, 16)]
                b1 = zv_v[r, pl.ds(16, 16)]
                p_v[r, pl.ds(0, 16)] = a0 * b0 + a1 * b1

            pltpu.sync_copy(p_v, out_h.at[pl.ds(off, CHUNK)])


    MB = 1000

    def mm1_body(x_r, w_r, o_r):
        o_r[...] = jnp.dot(x_r[...], w_r[...], preferred_element_type=f32)

    def norm_body(d_r, h_r, g_r, s_r):
        deg = d_r[0, :, 0:1] + d_r[1, :, 0:1] + 1.0
        dis = lax.rsqrt(deg)
        s_r[...] = dis
        g_r[...] = dis * h_r[...]

    def layer2_body(a_r, h_r, s_r, w_r, b_r, g_r, hw_r):
        dis = s_r[...]
        acc = a_r[0] + a_r[1]
        h = jnp.maximum(dis * acc + (dis * dis) * h_r[...] + b_r[...], 0.0)
        hw2 = jnp.dot(h, w_r[...], preferred_element_type=f32)
        hw_r[...] = hw2
        g_r[...] = dis * hw2

    def zout_body(a_r, h_r, s_r, b_r, z_r):
        dis = s_r[...]
        acc = a_r[0] + a_r[1]
        z_r[...] = dis * acc + (dis * dis) * h_r[...] + b_r[...]

    RB = 8000

    def red_body(p_r, o_r):
        o_r[...] = jnp.sum(p_r[...], axis=1, keepdims=True)


    degp = sc_deg(dst, zeros16, e0)

    hW1 = pl.pallas_call(
        mm1_body,
        grid=(N // MB,),
        in_specs=[
            pl.BlockSpec((MB, DIN), lambda i: (i, 0)),
            pl.BlockSpec((DIN, DH), lambda i: (0, 0)),
        ],
        out_specs=pl.BlockSpec((MB, DH), lambda i: (i, 0)),
        out_shape=jax.ShapeDtypeStruct((N, DH), f32),
    )(x, W1)

    g1, dis = pl.pallas_call(
        norm_body,
        grid=(N // MB,),
        in_specs=[
            pl.BlockSpec((NC, MB, 16), lambda i: (0, i, 0)),
            pl.BlockSpec((MB, DH), lambda i: (i, 0)),
        ],
        out_specs=[
            pl.BlockSpec((MB, DH), lambda i: (i, 0)),
            pl.BlockSpec((MB, 1), lambda i: (i, 0)),
        ],
        out_shape=[
            jax.ShapeDtypeStruct((N, DH), f32),
            jax.ShapeDtypeStruct((N, 1), f32),
        ],
    )(degp, hW1)

    acc1 = sc_seg(src, dst, g1, zeros32)

    g2, hW2 = pl.pallas_call(
        layer2_body,
        grid=(N // MB,),
        in_specs=[
            pl.BlockSpec((NC, MB, DH), lambda i: (0, i, 0)),
            pl.BlockSpec((MB, DH), lambda i: (i, 0)),
            pl.BlockSpec((MB, 1), lambda i: (i, 0)),
            pl.BlockSpec((DH, DO), lambda i: (0, 0)),
            pl.BlockSpec((1, DH), lambda i: (0, 0)),
        ],
        out_specs=[
            pl.BlockSpec((MB, DO), lambda i: (i, 0)),
            pl.BlockSpec((MB, DO), lambda i: (i, 0)),
        ],
        out_shape=[
            jax.ShapeDtypeStruct((N, DO), f32),
            jax.ShapeDtypeStruct((N, DO), f32),
        ],
    )(acc1, hW1, dis, W2, b1.reshape(1, DH))

    acc2 = sc_seg(src, dst, g2, zeros32)

    z = pl.pallas_call(
        zout_body,
        grid=(N // MB,),
        in_specs=[
            pl.BlockSpec((NC, MB, DO), lambda i: (0, i, 0)),
            pl.BlockSpec((MB, DO), lambda i: (i, 0)),
            pl.BlockSpec((MB, 1), lambda i: (i, 0)),
            pl.BlockSpec((1, DO), lambda i: (0, 0)),
        ],
        out_specs=pl.BlockSpec((MB, DO), lambda i: (i, 0)),
        out_shape=jax.ShapeDtypeStruct((N, DO), f32),
    )(acc2, hW2, dis, b2.reshape(1, DO))

    part = sc_dec(uu, vv, z)

    scores = pl.pallas_call(
        red_body,
        grid=(E // RB,),
        in_specs=[pl.BlockSpec((RB, 16), lambda i: (i, 0))],
        out_specs=pl.BlockSpec((RB, 1), lambda i: (i, 0)),
        out_shape=jax.ShapeDtypeStruct((E, 1), f32),
    )(part)

    return scores.reshape(E)


# speedup vs baseline: 9.3758x; 9.3758x over previous
import functools

import jax
import jax.numpy as jnp
from jax import lax
from jax.experimental import pallas as pl
from jax.experimental.pallas import tpu as pltpu
from jax.experimental.pallas import tpu_sc as plsc

NC = 2
NS = 16
NW = NC * NS
CHUNK = 80


def kernel(x, edge_index, edge_pairs, W1, b1, W2, b2):
    f32 = jnp.float32
    N0, DIN = x.shape
    DH = W1.shape[1]
    DO = W2.shape[1]
    E = edge_index.shape[1]

    N = ((N0 + 1023) // 1024) * 1024
    x = jnp.pad(x, ((0, N - N0), (0, 0)))

    EPT = E // NW
    NCH = EPT // CHUNK
    STR = N // NS

    src = edge_index[0].astype(jnp.int32)
    dst = edge_index[1].astype(jnp.int32)
    uu = edge_pairs[0].astype(jnp.int32)
    vv = edge_pairs[1].astype(jnp.int32)

    zeros16 = jnp.zeros((N, 16), f32)
    zeros32 = jnp.zeros((N, DH), f32)
    e0 = jnp.zeros((CHUNK, 16), f32).at[:, 0].set(1.0)

    mesh = plsc.VectorSubcoreMesh(core_axis_name="c", subcore_axis_name="s")
    sc_params = pltpu.CompilerParams(use_tc_tiling_on_sc=False)


    @functools.partial(
        pl.kernel,
        out_type=jax.ShapeDtypeStruct((NC, N, 16), f32),
        mesh=mesh,
        compiler_params=sc_params,
        scratch_types=[
            pltpu.VMEM((CHUNK,), jnp.int32),
            pltpu.VMEM((CHUNK, 16), f32),
            pltpu.VMEM_SHARED((N, 16), f32),
        ],
    )
    def sc_deg(dst_h, z_h, e0_h, out_h, idx_v, e0_v, acc_sh):
        cid = lax.axis_index("c")
        sid = lax.axis_index("s")
        base = (cid * NS + sid) * EPT
        sp = sid * STR
        pltpu.sync_copy(z_h.at[pl.ds(sp, STR)], acc_sh.at[pl.ds(sp, STR)])
        pltpu.sync_copy(e0_h, e0_v)
        plsc.subcore_barrier()

        @pl.loop(0, NCH)
        def _(k):
            pltpu.sync_copy(dst_h.at[pl.ds(base + k * CHUNK, CHUNK)], idx_v)
            pltpu.sync_copy(e0_v, acc_sh.at[idx_v], add=True)

        plsc.subcore_barrier()
        pltpu.sync_copy(acc_sh.at[pl.ds(sp, STR)], out_h.at[cid, pl.ds(sp, STR)])

    @functools.partial(
        pl.kernel,
        out_type=jax.ShapeDtypeStruct((NC, N, DH), f32),
        mesh=mesh,
        compiler_params=sc_params,
        scratch_types=[
            pltpu.VMEM((CHUNK,), jnp.int32),
            pltpu.VMEM((CHUNK,), jnp.int32),
            pltpu.VMEM((CHUNK, DH), f32),
            pltpu.VMEM_SHARED((N, DH), f32),
            pltpu.SemaphoreType.DMA,
        ],
    )
    def sc_seg(src_h, dst_h, tab_h, z_h, out_h, idx_s, idx_d, rows_v, acc_sh, sem):
        cid = lax.axis_index("c")
        sid = lax.axis_index("s")
        base = (cid * NS + sid) * EPT
        sp = sid * STR
        pltpu.sync_copy(z_h.at[pl.ds(sp, STR)], acc_sh.at[pl.ds(sp, STR)])
        plsc.subcore_barrier()

        @pl.loop(0, NCH)
        def _(k):
            off = base + k * CHUNK
            pltpu.sync_copy(src_h.at[pl.ds(off, CHUNK)], idx_s)
            pltpu.async_copy(tab_h.at[idx_s], rows_v, sem).wait()
            pltpu.sync_copy(dst_h.at[pl.ds(off, CHUNK)], idx_d)
            pltpu.sync_copy(rows_v, acc_sh.at[idx_d], add=True)

        plsc.subcore_barrier()
        pltpu.sync_copy(acc_sh.at[pl.ds(sp, STR)], out_h.at[cid, pl.ds(sp, STR)])

    @functools.partial(
        pl.kernel,
        out_type=jax.ShapeDtypeStruct((E, 16), f32),
        mesh=mesh,
        compiler_params=sc_params,
        scratch_types=[
            pltpu.VMEM((CHUNK,), jnp.int32),
            pltpu.VMEM((CHUNK,), jnp.int32),
            pltpu.VMEM((CHUNK, DO), f32),
            pltpu.VMEM((CHUNK, DO), f32),
            pltpu.VMEM((CHUNK, 16), f32),
            pltpu.SemaphoreType.DMA,
        ],
    )
    def sc_dec(u_h, v_h, z_h, out_h, iu_v, iv_v, zu_v, zv_v, p_v, sem):
        cid = lax.axis_index("c")
        sid = lax.axis_index("s")
        base = (cid * NS + sid) * EPT

        @pl.loop(0, NCH)
        def _(k):
            off = base + k * CHUNK
            pltpu.sync_copy(u_h.at[pl.ds(off, CHUNK)], iu_v)
            pltpu.sync_copy(v_h.at[pl.ds(off, CHUNK)], iv_v)
            cp1 = pltpu.async_copy(z_h.at[iu_v], zu_v, sem)
            cp2 = pltpu.async_copy(z_h.at[iv_v], zv_v, sem)
            cp1.wait()
            cp2.wait()

            @pl.loop(0, CHUNK)
            def _(r):
                a0 = zu_v[r, pl.ds(0, 16)]
                a1 = zu_v[r, pl.ds(16, 16)]
                b0 = zv_v[r, pl.ds(0, 16)]
                b1 = zv_v[r, pl.ds(16, 16)]
                p_v[r, pl.ds(0, 16)] = a0 * b0 + a1 * b1

            pltpu.sync_copy(p_v, out_h.at[pl.ds(off, CHUNK)])


    MB = N // 8

    def mm1_body(x_r, w_r, o_r):
        o_r[...] = jnp.dot(x_r[...], w_r[...], preferred_element_type=f32)

    def norm_body(d_r, h_r, g_r, s_r):
        deg = d_r[0, :, 0:1] + d_r[1, :, 0:1] + 1.0
        dis = lax.rsqrt(deg)
        s_r[...] = dis
        g_r[...] = dis * h_r[...]

    def layer2_body(a_r, h_r, s_r, w_r, b_r, g_r, hw_r):
        dis = s_r[...]
        acc = a_r[0] + a_r[1]
        h = jnp.maximum(dis * acc + (dis * dis) * h_r[...] + b_r[...], 0.0)
        hw2 = jnp.dot(h, w_r[...], preferred_element_type=f32)
        hw_r[...] = hw2
        g_r[...] = dis * hw2

    def zout_body(a_r, h_r, s_r, b_r, z_r):
        dis = s_r[...]
        acc = a_r[0] + a_r[1]
        z_r[...] = dis * acc + (dis * dis) * h_r[...] + b_r[...]

    RB = 8000

    def red_body(p_r, o_r):
        o_r[...] = jnp.sum(p_r[...], axis=1, keepdims=True)


    degp = sc_deg(dst, zeros16, e0)

    hW1 = pl.pallas_call(
        mm1_body,
        grid=(N // MB,),
        in_specs=[
            pl.BlockSpec((MB, DIN), lambda i: (i, 0)),
            pl.BlockSpec((DIN, DH), lambda i: (0, 0)),
        ],
        out_specs=pl.BlockSpec((MB, DH), lambda i: (i, 0)),
        out_shape=jax.ShapeDtypeStruct((N, DH), f32),
    )(x, W1)

    g1, dis = pl.pallas_call(
        norm_body,
        grid=(N // MB,),
        in_specs=[
            pl.BlockSpec((NC, MB, 16), lambda i: (0, i, 0)),
            pl.BlockSpec((MB, DH), lambda i: (i, 0)),
        ],
        out_specs=[
            pl.BlockSpec((MB, DH), lambda i: (i, 0)),
            pl.BlockSpec((MB, 1), lambda i: (i, 0)),
        ],
        out_shape=[
            jax.ShapeDtypeStruct((N, DH), f32),
            jax.ShapeDtypeStruct((N, 1), f32),
        ],
    )(degp, hW1)

    acc1 = sc_seg(src, dst, g1, zeros32)

    g2, hW2 = pl.pallas_call(
        layer2_body,
        grid=(N // MB,),
        in_specs=[
            pl.BlockSpec((NC, MB, DH), lambda i: (0, i, 0)),
            pl.BlockSpec((MB, DH), lambda i: (i, 0)),
            pl.BlockSpec((MB, 1), lambda i: (i, 0)),
            pl.BlockSpec((DH, DO), lambda i: (0, 0)),
            pl.BlockSpec((1, DH), lambda i: (0, 0)),
        ],
        out_specs=[
            pl.BlockSpec((MB, DO), lambda i: (i, 0)),
            pl.BlockSpec((MB, DO), lambda i: (i, 0)),
        ],
        out_shape=[
            jax.ShapeDtypeStruct((N, DO), f32),
            jax.ShapeDtypeStruct((N, DO), f32),
        ],
    )(acc1, hW1, dis, W2, b1.reshape(1, DH))

    acc2 = sc_seg(src, dst, g2, zeros32)

    z = pl.pallas_call(
        zout_body,
        grid=(N // MB,),
        in_specs=[
            pl.BlockSpec((NC, MB, DO), lambda i: (0, i, 0)),
            pl.BlockSpec((MB, DO), lambda i: (i, 0)),
            pl.BlockSpec((MB, 1), lambda i: (i, 0)),
            pl.BlockSpec((1, DO), lambda i: (0, 0)),
        ],
        out_specs=pl.BlockSpec((MB, DO), lambda i: (i, 0)),
        out_shape=jax.ShapeDtypeStruct((N, DO), f32),
    )(acc2, hW2, dis, b2.reshape(1, DO))

    part = sc_dec(uu, vv, z)

    scores = pl.pallas_call(
        red_body,
        grid=(E // RB,),
        in_specs=[pl.BlockSpec((RB, 16), lambda i: (i, 0))],
        out_specs=pl.BlockSpec((RB, 1), lambda i: (i, 0)),
        out_shape=jax.ShapeDtypeStruct((E, 1), f32),
    )(part)

    return scores.reshape(E)


# speedup vs baseline: 20.0484x; 2.1383x over previous
import functools

import jax
import jax.numpy as jnp
from jax import lax
from jax.experimental import pallas as pl
from jax.experimental.pallas import tpu as pltpu
from jax.experimental.pallas import tpu_sc as plsc

NC = 2
NS = 16
NW = NC * NS
CHUNK = 80
NBUF = 10
NSET = 3


def kernel(x, edge_index, edge_pairs, W1, b1, W2, b2):
    f32 = jnp.float32
    N0, DIN = x.shape
    DH = W1.shape[1]
    DO = W2.shape[1]
    E = edge_index.shape[1]

    N = ((N0 + 1023) // 1024) * 1024
    x = jnp.pad(x, ((0, N - N0), (0, 0)))

    EPT = E // NW
    NCH = EPT // CHUNK
    STR = N // NS

    src2 = edge_index[0].astype(jnp.int32).reshape(E // CHUNK, CHUNK)
    dst2 = edge_index[1].astype(jnp.int32).reshape(E // CHUNK, CHUNK)
    uu2 = edge_pairs[0].astype(jnp.int32).reshape(E // CHUNK, CHUNK)
    vv2 = edge_pairs[1].astype(jnp.int32).reshape(E // CHUNK, CHUNK)

    zeros16 = jnp.zeros((N, 16), f32)
    zeros32 = jnp.zeros((N, DH), f32)
    e0 = jnp.zeros((CHUNK, 16), f32).at[:, 0].set(1.0)

    mesh = plsc.VectorSubcoreMesh(core_axis_name="c", subcore_axis_name="s")
    sc_params = pltpu.CompilerParams(use_tc_tiling_on_sc=False)


    @functools.partial(
        pl.kernel,
        out_type=jax.ShapeDtypeStruct((NC, N, 16), f32),
        mesh=mesh,
        compiler_params=sc_params,
        scratch_types=[
            pltpu.VMEM((NCH, CHUNK), jnp.int32),
            pltpu.VMEM((CHUNK, 16), f32),
            pltpu.VMEM_SHARED((N, 16), f32),
            pltpu.SemaphoreType.DMA,
        ],
    )
    def sc_deg(dst_h, z_h, e0_h, out_h, didx_v, e0_v, acc_sh, sem):
        cid = lax.axis_index("c")
        sid = lax.axis_index("s")
        rb = (cid * NS + sid) * NCH
        sp = sid * STR
        pltpu.sync_copy(z_h.at[pl.ds(sp, STR)], acc_sh.at[pl.ds(sp, STR)])
        pltpu.sync_copy(dst_h.at[pl.ds(rb, NCH)], didx_v)
        pltpu.sync_copy(e0_h, e0_v)
        plsc.subcore_barrier()

        @pl.loop(0, NCH)
        def _(k):
            pltpu.async_copy(e0_v, acc_sh.at[didx_v.at[k]], sem, add=True)

        @pl.loop(0, NCH)
        def _(k):
            pltpu.make_async_copy(e0_v, acc_sh.at[didx_v.at[0]], sem).wait()

        plsc.subcore_barrier()
        pltpu.sync_copy(acc_sh.at[pl.ds(sp, STR)], out_h.at[cid, pl.ds(sp, STR)])

    @functools.partial(
        pl.kernel,
        out_type=jax.ShapeDtypeStruct((NC, N, DH), f32),
        mesh=mesh,
        compiler_params=sc_params,
        scratch_types=[
            pltpu.VMEM((NCH, CHUNK), jnp.int32),
            pltpu.VMEM((NCH, CHUNK), jnp.int32),
            pltpu.VMEM((NBUF, CHUNK, DH), f32),
            pltpu.VMEM_SHARED((N, DH), f32),
            pltpu.VMEM_SHARED((N, DH), f32),
            pltpu.SemaphoreType.DMA((NBUF,)),
            pltpu.SemaphoreType.DMA((NBUF,)),
        ],
    )
    def sc_seg(src_h, dst_h, tab_h, z_h, out_h,
               sidx_v, didx_v, rows_r, tab_sh, acc_sh, semg, sems):
        cid = lax.axis_index("c")
        sid = lax.axis_index("s")
        rb = (cid * NS + sid) * NCH
        sp = sid * STR
        pltpu.sync_copy(z_h.at[pl.ds(sp, STR)], acc_sh.at[pl.ds(sp, STR)])
        pltpu.sync_copy(tab_h.at[pl.ds(sp, STR)], tab_sh.at[pl.ds(sp, STR)])
        pltpu.sync_copy(src_h.at[pl.ds(rb, NCH)], sidx_v)
        pltpu.sync_copy(dst_h.at[pl.ds(rb, NCH)], didx_v)
        plsc.subcore_barrier()

        for b in range(NBUF):
            pltpu.async_copy(tab_sh.at[sidx_v.at[b]], rows_r.at[b], semg.at[b])

        @pl.loop(0, NCH + NBUF - (NCH % NBUF), step=NBUF)
        def _(k):
            for b in range(NBUF):
                c = k + b

                @pl.when(c < NCH)
                def _():
                    pltpu.make_async_copy(
                        tab_sh.at[sidx_v.at[0]], rows_r.at[b], semg.at[b]
                    ).wait()
                    pltpu.async_copy(
                        rows_r.at[b], acc_sh.at[didx_v.at[c]], sems.at[b],
                        add=True,
                    )

            for b in range(NBUF):
                c = k + b

                @pl.when(c < NCH)
                def _():
                    pltpu.make_async_copy(
                        rows_r.at[b], acc_sh.at[didx_v.at[0]], sems.at[b]
                    ).wait()

                @pl.when(c + NBUF < NCH)
                def _():
                    pltpu.async_copy(
                        tab_sh.at[sidx_v.at[c + NBUF]], rows_r.at[b],
                        semg.at[b],
                    )

        plsc.subcore_barrier()
        pltpu.sync_copy(acc_sh.at[pl.ds(sp, STR)], out_h.at[cid, pl.ds(sp, STR)])

    @functools.partial(
        pl.kernel,
        out_type=jax.ShapeDtypeStruct((E, 16), f32),
        mesh=mesh,
        compiler_params=sc_params,
        scratch_types=[
            pltpu.VMEM((NCH, CHUNK), jnp.int32),
            pltpu.VMEM((NCH, CHUNK), jnp.int32),
            pltpu.VMEM((NSET, CHUNK, DO), f32),
            pltpu.VMEM((NSET, CHUNK, DO), f32),
            pltpu.VMEM((NSET, CHUNK, 16), f32),
            pltpu.VMEM_SHARED((N, DO), f32),
            pltpu.SemaphoreType.DMA((NSET,)),
            pltpu.SemaphoreType.DMA((NSET,)),
            pltpu.SemaphoreType.DMA((NSET,)),
        ],
    )
    def sc_dec(u_h, v_h, z_h, out_h,
               uix_v, vix_v, zu_r, zv_r, p_r, z_sh, semu, semv, semp):
        cid = lax.axis_index("c")
        sid = lax.axis_index("s")
        g = cid * NS + sid
        rb = g * NCH
        base = g * EPT
        sp = sid * STR
        pltpu.sync_copy(z_h.at[pl.ds(sp, STR)], z_sh.at[pl.ds(sp, STR)])
        pltpu.sync_copy(u_h.at[pl.ds(rb, NCH)], uix_v)
        pltpu.sync_copy(v_h.at[pl.ds(rb, NCH)], vix_v)
        plsc.subcore_barrier()

        for s in range(NSET):
            pltpu.async_copy(z_sh.at[uix_v.at[s]], zu_r.at[s], semu.at[s])
            pltpu.async_copy(z_sh.at[vix_v.at[s]], zv_r.at[s], semv.at[s])

        @pl.loop(0, NCH + NSET - (NCH % NSET), step=NSET)
        def _(k):
            for s in range(NSET):
                c = k + s

                @pl.when(c < NCH)
                def _():
                    pltpu.make_async_copy(
                        z_sh.at[uix_v.at[0]], zu_r.at[s], semu.at[s]
                    ).wait()
                    pltpu.make_async_copy(
                        z_sh.at[vix_v.at[0]], zv_r.at[s], semv.at[s]
                    ).wait()

                    @pl.when(c >= NSET)
                    def _():
                        pltpu.make_async_copy(
                            p_r.at[s], out_h.at[pl.ds(0, CHUNK)], semp.at[s]
                        ).wait()

                    @pl.loop(0, CHUNK)
                    def _(r):
                        a0 = zu_r[s, r, pl.ds(0, 16)]
                        a1 = zu_r[s, r, pl.ds(16, 16)]
                        b0 = zv_r[s, r, pl.ds(0, 16)]
                        b1 = zv_r[s, r, pl.ds(16, 16)]
                        p_r[s, r, pl.ds(0, 16)] = a0 * b0 + a1 * b1

                    pltpu.async_copy(
                        p_r.at[s], out_h.at[pl.ds(base + c * CHUNK, CHUNK)],
                        semp.at[s],
                    )

                    @pl.when(c + NSET < NCH)
                    def _():
                        pltpu.async_copy(
                            z_sh.at[uix_v.at[c + NSET]], zu_r.at[s], semu.at[s]
                        )
                        pltpu.async_copy(
                            z_sh.at[vix_v.at[c + NSET]], zv_r.at[s], semv.at[s]
                        )

        for s in range(NSET):
            pltpu.make_async_copy(
                p_r.at[s], out_h.at[pl.ds(0, CHUNK)], semp.at[s]
            ).wait()


    MB = N // 8

    def mm1_body(x_r, w_r, o_r):
        o_r[...] = jnp.dot(x_r[...], w_r[...], preferred_element_type=f32)

    def norm_body(d_r, h_r, g_r, s_r):
        deg = d_r[0, :, 0:1] + d_r[1, :, 0:1] + 1.0
        dis = lax.rsqrt(deg)
        s_r[...] = dis
        g_r[...] = dis * h_r[...]

    def layer2_body(a_r, h_r, s_r, w_r, b_r, g_r, hw_r):
        dis = s_r[...]
        acc = a_r[0] + a_r[1]
        h = jnp.maximum(dis * acc + (dis * dis) * h_r[...] + b_r[...], 0.0)
        hw2 = jnp.dot(h, w_r[...], preferred_element_type=f32)
        hw_r[...] = hw2
        g_r[...] = dis * hw2

    def zout_body(a_r, h_r, s_r, b_r, z_r):
        dis = s_r[...]
        acc = a_r[0] + a_r[1]
        z_r[...] = dis * acc + (dis * dis) * h_r[...] + b_r[...]

    RB = 8000

    def red_body(p_r, o_r):
        o_r[...] = jnp.sum(p_r[...], axis=1, keepdims=True)


    degp = sc_deg(dst2, zeros16, e0)

    hW1 = pl.pallas_call(
        mm1_body,
        grid=(N // MB,),
        in_specs=[
            pl.BlockSpec((MB, DIN), lambda i: (i, 0)),
            pl.BlockSpec((DIN, DH), lambda i: (0, 0)),
        ],
        out_specs=pl.BlockSpec((MB, DH), lambda i: (i, 0)),
        out_shape=jax.ShapeDtypeStruct((N, DH), f32),
    )(x, W1)

    g1, dis = pl.pallas_call(
        norm_body,
        grid=(N // MB,),
        in_specs=[
            pl.BlockSpec((NC, MB, 16), lambda i: (0, i, 0)),
            pl.BlockSpec((MB, DH), lambda i: (i, 0)),
        ],
        out_specs=[
            pl.BlockSpec((MB, DH), lambda i: (i, 0)),
            pl.BlockSpec((MB, 1), lambda i: (i, 0)),
        ],
        out_shape=[
            jax.ShapeDtypeStruct((N, DH), f32),
            jax.ShapeDtypeStruct((N, 1), f32),
        ],
    )(degp, hW1)

    acc1 = sc_seg(src2, dst2, g1, zeros32)

    g2, hW2 = pl.pallas_call(
        layer2_body,
        grid=(N // MB,),
        in_specs=[
            pl.BlockSpec((NC, MB, DH), lambda i: (0, i, 0)),
            pl.BlockSpec((MB, DH), lambda i: (i, 0)),
            pl.BlockSpec((MB, 1), lambda i: (i, 0)),
            pl.BlockSpec((DH, DO), lambda i: (0, 0)),
            pl.BlockSpec((1, DH), lambda i: (0, 0)),
        ],
        out_specs=[
            pl.BlockSpec((MB, DO), lambda i: (i, 0)),
            pl.BlockSpec((MB, DO), lambda i: (i, 0)),
        ],
        out_shape=[
            jax.ShapeDtypeStruct((N, DO), f32),
            jax.ShapeDtypeStruct((N, DO), f32),
        ],
    )(acc1, hW1, dis, W2, b1.reshape(1, DH))

    acc2 = sc_seg(src2, dst2, g2, zeros32)

    z = pl.pallas_call(
        zout_body,
        grid=(N // MB,),
        in_specs=[
            pl.BlockSpec((NC, MB, DO), lambda i: (0, i, 0)),
            pl.BlockSpec((MB, DO), lambda i: (i, 0)),
            pl.BlockSpec((MB, 1), lambda i: (i, 0)),
            pl.BlockSpec((1, DO), lambda i: (0, 0)),
        ],
        out_specs=pl.BlockSpec((MB, DO), lambda i: (i, 0)),
        out_shape=jax.ShapeDtypeStruct((N, DO), f32),
    )(acc2, hW2, dis, b2.reshape(1, DO))

    part = sc_dec(uu2, vv2, z)

    scores = pl.pallas_call(
        red_body,
        grid=(E // RB,),
        in_specs=[pl.BlockSpec((RB, 16), lambda i: (i, 0))],
        out_specs=pl.BlockSpec((RB, 1), lambda i: (i, 0)),
        out_shape=jax.ShapeDtypeStruct((E, 1), f32),
    )(part)

    return scores.reshape(E)


# speedup vs baseline: 30.7363x; 1.5331x over previous
import functools

import jax
import jax.numpy as jnp
from jax import lax
from jax.experimental import pallas as pl
from jax.experimental.pallas import tpu as pltpu
from jax.experimental.pallas import tpu_sc as plsc

NC = 2
NS = 16
NW = NC * NS
CHUNK = 80
NBUF = 10
NSET = 3


def kernel(x, edge_index, edge_pairs, W1, b1, W2, b2):
    f32 = jnp.float32
    N0, DIN = x.shape
    DH = W1.shape[1]
    DO = W2.shape[1]
    E = edge_index.shape[1]

    N = ((N0 + 1023) // 1024) * 1024
    x = jnp.pad(x, ((0, N - N0), (0, 0)))

    EPT = E // NW
    NCH = EPT // CHUNK
    STR = N // NS

    src2 = edge_index[0].astype(jnp.int32).reshape(E // CHUNK, CHUNK)
    dst2 = edge_index[1].astype(jnp.int32).reshape(E // CHUNK, CHUNK)
    uu2 = edge_pairs[0].astype(jnp.int32).reshape(E // CHUNK, CHUNK)
    vv2 = edge_pairs[1].astype(jnp.int32).reshape(E // CHUNK, CHUNK)

    zeros16 = jnp.zeros((N, 16), f32)
    zeros32 = jnp.zeros((N, DH), f32)
    e0 = jnp.zeros((CHUNK, 16), f32).at[:, 0].set(1.0)

    mesh = plsc.VectorSubcoreMesh(core_axis_name="c", subcore_axis_name="s")
    sc_params = pltpu.CompilerParams(use_tc_tiling_on_sc=False)
    sc_params_nl = pltpu.CompilerParams(
        use_tc_tiling_on_sc=False, needs_layout_passes=False
    )


    @functools.partial(
        pl.kernel,
        out_type=jax.ShapeDtypeStruct((NC, N, 16), f32),
        mesh=mesh,
        compiler_params=sc_params,
        scratch_types=[
            pltpu.VMEM((NCH, CHUNK), jnp.int32),
            pltpu.VMEM((CHUNK, 16), f32),
            pltpu.VMEM_SHARED((N, 16), f32),
            pltpu.SemaphoreType.DMA,
        ],
    )
    def sc_deg(dst_h, z_h, e0_h, out_h, didx_v, e0_v, acc_sh, sem):
        cid = lax.axis_index("c")
        sid = lax.axis_index("s")
        rb = (cid * NS + sid) * NCH
        sp = sid * STR
        pltpu.sync_copy(z_h.at[pl.ds(sp, STR)], acc_sh.at[pl.ds(sp, STR)])
        pltpu.sync_copy(dst_h.at[pl.ds(rb, NCH)], didx_v)
        pltpu.sync_copy(e0_h, e0_v)
        plsc.subcore_barrier()

        @pl.loop(0, NCH)
        def _(k):
            pltpu.async_copy(e0_v, acc_sh.at[didx_v.at[k]], sem, add=True)

        @pl.loop(0, NCH)
        def _(k):
            pltpu.make_async_copy(e0_v, acc_sh.at[didx_v.at[0]], sem).wait()

        plsc.subcore_barrier()
        pltpu.sync_copy(acc_sh.at[pl.ds(sp, STR)], out_h.at[cid, pl.ds(sp, STR)])

    @functools.partial(
        pl.kernel,
        out_type=jax.ShapeDtypeStruct((NC, N, DH), f32),
        mesh=mesh,
        compiler_params=sc_params,
        scratch_types=[
            pltpu.VMEM((NCH, CHUNK), jnp.int32),
            pltpu.VMEM((NCH, CHUNK), jnp.int32),
            pltpu.VMEM((NBUF, CHUNK, DH), f32),
            pltpu.VMEM_SHARED((N, DH), f32),
            pltpu.VMEM_SHARED((N, DH), f32),
            pltpu.SemaphoreType.DMA((NBUF,)),
            pltpu.SemaphoreType.DMA((NBUF,)),
        ],
    )
    def sc_seg(src_h, dst_h, tab_h, z_h, out_h,
               sidx_v, didx_v, rows_r, tab_sh, acc_sh, semg, sems):
        cid = lax.axis_index("c")
        sid = lax.axis_index("s")
        rb = (cid * NS + sid) * NCH
        sp = sid * STR
        pltpu.sync_copy(z_h.at[pl.ds(sp, STR)], acc_sh.at[pl.ds(sp, STR)])
        pltpu.sync_copy(tab_h.at[pl.ds(sp, STR)], tab_sh.at[pl.ds(sp, STR)])
        pltpu.sync_copy(src_h.at[pl.ds(rb, NCH)], sidx_v)
        pltpu.sync_copy(dst_h.at[pl.ds(rb, NCH)], didx_v)
        plsc.subcore_barrier()

        for b in range(NBUF):
            pltpu.async_copy(tab_sh.at[sidx_v.at[b]], rows_r.at[b], semg.at[b])

        @pl.loop(0, NCH + NBUF - (NCH % NBUF), step=NBUF)
        def _(k):
            for b in range(NBUF):
                c = k + b

                @pl.when(c < NCH)
                def _():
                    pltpu.make_async_copy(
                        tab_sh.at[sidx_v.at[0]], rows_r.at[b], semg.at[b]
                    ).wait()
                    pltpu.async_copy(
                        rows_r.at[b], acc_sh.at[didx_v.at[c]], sems.at[b],
                        add=True,
                    )

            for b in range(NBUF):
                c = k + b

                @pl.when(c < NCH)
                def _():
                    pltpu.make_async_copy(
                        rows_r.at[b], acc_sh.at[didx_v.at[0]], sems.at[b]
                    ).wait()

                @pl.when(c + NBUF < NCH)
                def _():
                    pltpu.async_copy(
                        tab_sh.at[sidx_v.at[c + NBUF]], rows_r.at[b],
                        semg.at[b],
                    )

        plsc.subcore_barrier()
        pltpu.sync_copy(acc_sh.at[pl.ds(sp, STR)], out_h.at[cid, pl.ds(sp, STR)])

    @functools.partial(
        pl.kernel,
        out_type=jax.ShapeDtypeStruct((E,), f32),
        mesh=mesh,
        compiler_params=sc_params_nl,
        scratch_types=[
            pltpu.VMEM((NCH, CHUNK), jnp.int32),
            pltpu.VMEM((NCH, CHUNK), jnp.int32),
            pltpu.VMEM((NSET, CHUNK, DO), f32),
            pltpu.VMEM((NSET, CHUNK, DO), f32),
            pltpu.VMEM((NSET * (CHUNK + 16),), f32),
            pltpu.VMEM_SHARED((N, DO), f32),
            pltpu.SemaphoreType.DMA((NSET,)),
            pltpu.SemaphoreType.DMA((NSET,)),
            pltpu.SemaphoreType.DMA((NSET,)),
        ],
    )
    def sc_dec(u_h, v_h, z_h, out_h,
               uix_v, vix_v, zu_r, zv_r, p_r, z_sh, semu, semv, semp):
        cid = lax.axis_index("c")
        sid = lax.axis_index("s")
        g = cid * NS + sid
        rb = g * NCH
        base = g * EPT
        sp = sid * STR
        lane15 = jax.lax.iota(jnp.int32, 16) == 15
        pltpu.sync_copy(z_h.at[pl.ds(sp, STR)], z_sh.at[pl.ds(sp, STR)])
        pltpu.sync_copy(u_h.at[pl.ds(rb, NCH)], uix_v)
        pltpu.sync_copy(v_h.at[pl.ds(rb, NCH)], vix_v)
        plsc.subcore_barrier()

        for s in range(NSET):
            pltpu.async_copy(z_sh.at[uix_v.at[s]], zu_r.at[s], semu.at[s])
            pltpu.async_copy(z_sh.at[vix_v.at[s]], zv_r.at[s], semv.at[s])

        @pl.loop(0, NCH + NSET - (NCH % NSET), step=NSET)
        def _(k):
            for s in range(NSET):
                c = k + s

                @pl.when(c < NCH)
                def _():
                    pltpu.make_async_copy(
                        z_sh.at[uix_v.at[0]], zu_r.at[s], semu.at[s]
                    ).wait()
                    pltpu.make_async_copy(
                        z_sh.at[vix_v.at[0]], zv_r.at[s], semv.at[s]
                    ).wait()

                    @pl.when(c >= NSET)
                    def _():
                        pltpu.make_async_copy(
                            p_r.at[pl.ds(s * (CHUNK + 16), CHUNK)],
                            out_h.at[pl.ds(0, CHUNK)], semp.at[s]
                        ).wait()

                    @pl.loop(0, CHUNK, step=4)
                    def _(r):
                        for j in range(4):
                            a0 = zu_r[s, r + j, pl.ds(0, 16)]
                            a1 = zu_r[s, r + j, pl.ds(16, 16)]
                            b0 = zv_r[s, r + j, pl.ds(0, 16)]
                            b1 = zv_r[s, r + j, pl.ds(16, 16)]
                            cum = plsc.cumsum(a0 * b0 + a1 * b1)
                            plsc.store_compressed(
                                p_r.at[pl.ds(s * (CHUNK + 16) + r + j, 16)],
                                cum, mask=lane15,
                            )

                    pltpu.async_copy(
                        p_r.at[pl.ds(s * (CHUNK + 16), CHUNK)],
                        out_h.at[pl.ds(base + c * CHUNK, CHUNK)],
                        semp.at[s],
                    )

                    @pl.when(c + NSET < NCH)
                    def _():
                        pltpu.async_copy(
                            z_sh.at[uix_v.at[c + NSET]], zu_r.at[s], semu.at[s]
                        )
                        pltpu.async_copy(
                            z_sh.at[vix_v.at[c + NSET]], zv_r.at[s], semv.at[s]
                        )

        for s in range(NSET):
            pltpu.make_async_copy(
                p_r.at[pl.ds(s * (CHUNK + 16), CHUNK)],
                out_h.at[pl.ds(0, CHUNK)], semp.at[s]
            ).wait()


    MB = N // 8

    def mm1_body(x_r, w_r, o_r):
        o_r[...] = jnp.dot(x_r[...], w_r[...], preferred_element_type=f32)

    def norm_body(d_r, h_r, g_r, s_r):
        deg = d_r[0, :, 0:1] + d_r[1, :, 0:1] + 1.0
        dis = lax.rsqrt(deg)
        s_r[...] = dis
        g_r[...] = dis * h_r[...]

    def layer2_body(a_r, h_r, s_r, w_r, b_r, g_r, hw_r):
        dis = s_r[...]
        acc = a_r[0] + a_r[1]
        h = jnp.maximum(dis * acc + (dis * dis) * h_r[...] + b_r[...], 0.0)
        hw2 = jnp.dot(h, w_r[...], preferred_element_type=f32)
        hw_r[...] = hw2
        g_r[...] = dis * hw2

    def zout_body(a_r, h_r, s_r, b_r, z_r):
        dis = s_r[...]
        acc = a_r[0] + a_r[1]
        z_r[...] = dis * acc + (dis * dis) * h_r[...] + b_r[...]


    degp = sc_deg(dst2, zeros16, e0)

    hW1 = pl.pallas_call(
        mm1_body,
        grid=(N // MB,),
        in_specs=[
            pl.BlockSpec((MB, DIN), lambda i: (i, 0)),
            pl.BlockSpec((DIN, DH), lambda i: (0, 0)),
        ],
        out_specs=pl.BlockSpec((MB, DH), lambda i: (i, 0)),
        out_shape=jax.ShapeDtypeStruct((N, DH), f32),
    )(x, W1)

    g1, dis = pl.pallas_call(
        norm_body,
        grid=(N // MB,),
        in_specs=[
            pl.BlockSpec((NC, MB, 16), lambda i: (0, i, 0)),
            pl.BlockSpec((MB, DH), lambda i: (i, 0)),
        ],
        out_specs=[
            pl.BlockSpec((MB, DH), lambda i: (i, 0)),
            pl.BlockSpec((MB, 1), lambda i: (i, 0)),
        ],
        out_shape=[
            jax.ShapeDtypeStruct((N, DH), f32),
            jax.ShapeDtypeStruct((N, 1), f32),
        ],
    )(degp, hW1)

    acc1 = sc_seg(src2, dst2, g1, zeros32)

    g2, hW2 = pl.pallas_call(
        layer2_body,
        grid=(N // MB,),
        in_specs=[
            pl.BlockSpec((NC, MB, DH), lambda i: (0, i, 0)),
            pl.BlockSpec((MB, DH), lambda i: (i, 0)),
            pl.BlockSpec((MB, 1), lambda i: (i, 0)),
            pl.BlockSpec((DH, DO), lambda i: (0, 0)),
            pl.BlockSpec((1, DH), lambda i: (0, 0)),
        ],
        out_specs=[
            pl.BlockSpec((MB, DO), lambda i: (i, 0)),
            pl.BlockSpec((MB, DO), lambda i: (i, 0)),
        ],
        out_shape=[
            jax.ShapeDtypeStruct((N, DO), f32),
            jax.ShapeDtypeStruct((N, DO), f32),
        ],
    )(acc1, hW1, dis, W2, b1.reshape(1, DH))

    acc2 = sc_seg(src2, dst2, g2, zeros32)

    z = pl.pallas_call(
        zout_body,
        grid=(N // MB,),
        in_specs=[
            pl.BlockSpec((NC, MB, DO), lambda i: (0, i, 0)),
            pl.BlockSpec((MB, DO), lambda i: (i, 0)),
            pl.BlockSpec((MB, 1), lambda i: (i, 0)),
            pl.BlockSpec((1, DO), lambda i: (0, 0)),
        ],
        out_specs=pl.BlockSpec((MB, DO), lambda i: (i, 0)),
        out_shape=jax.ShapeDtypeStruct((N, DO), f32),
    )(acc2, hW2, dis, b2.reshape(1, DO))

    return sc_dec(uu2, vv2, z)


# speedup vs baseline: 42.4138x; 1.3799x over previous
import functools

import jax
import jax.numpy as jnp
from jax import lax
from jax.experimental import pallas as pl
from jax.experimental.pallas import tpu as pltpu
from jax.experimental.pallas import tpu_sc as plsc

NC = 2
NS = 16
NW = NC * NS
CHUNK = 80
NBUF = 10
NSET = 3


def kernel(x, edge_index, edge_pairs, W1, b1, W2, b2):
    f32 = jnp.float32
    N0, DIN = x.shape
    DH = W1.shape[1]
    DO = W2.shape[1]
    E = edge_index.shape[1]

    N = ((N0 + 1023) // 1024) * 1024
    x = jnp.pad(x, ((0, N - N0), (0, 0)))

    EPT = E // NW
    NCH = EPT // CHUNK
    STR = N // NS

    src2 = edge_index[0].astype(jnp.int32).reshape(E // CHUNK, CHUNK)
    dst2 = edge_index[1].astype(jnp.int32).reshape(E // CHUNK, CHUNK)
    uu2 = edge_pairs[0].astype(jnp.int32).reshape(E // CHUNK, CHUNK)
    vv2 = edge_pairs[1].astype(jnp.int32).reshape(E // CHUNK, CHUNK)

    zeros16 = jnp.zeros((N, 16), f32)
    zeros32 = jnp.zeros((N, DH), f32)
    e0 = jnp.zeros((CHUNK, 16), f32).at[:, 0].set(1.0)

    mesh = plsc.VectorSubcoreMesh(core_axis_name="c", subcore_axis_name="s")
    sc_params = pltpu.CompilerParams(use_tc_tiling_on_sc=False)
    sc_params_nl = pltpu.CompilerParams(
        use_tc_tiling_on_sc=False, needs_layout_passes=False
    )


    @functools.partial(
        pl.kernel,
        out_type=jax.ShapeDtypeStruct((NC, N, 16), f32),
        mesh=mesh,
        compiler_params=sc_params,
        scratch_types=[
            pltpu.VMEM((NCH, CHUNK), jnp.int32),
            pltpu.VMEM((CHUNK, 16), f32),
            pltpu.VMEM_SHARED((N, 16), f32),
            pltpu.SemaphoreType.DMA,
        ],
    )
    def sc_deg(dst_h, z_h, e0_h, out_h, didx_v, e0_v, acc_sh, sem):
        cid = lax.axis_index("c")
        sid = lax.axis_index("s")
        rb = (cid * NS + sid) * NCH
        sp = sid * STR
        pltpu.sync_copy(z_h.at[pl.ds(sp, STR)], acc_sh.at[pl.ds(sp, STR)])
        pltpu.sync_copy(dst_h.at[pl.ds(rb, NCH)], didx_v)
        pltpu.sync_copy(e0_h, e0_v)
        plsc.subcore_barrier()

        @pl.loop(0, NCH)
        def _(k):
            pltpu.async_copy(e0_v, acc_sh.at[didx_v.at[k]], sem, add=True)

        @pl.loop(0, NCH)
        def _(k):
            pltpu.make_async_copy(e0_v, acc_sh.at[didx_v.at[0]], sem).wait()

        plsc.subcore_barrier()
        pltpu.sync_copy(acc_sh.at[pl.ds(sp, STR)], out_h.at[cid, pl.ds(sp, STR)])

    @functools.partial(
        pl.kernel,
        out_type=jax.ShapeDtypeStruct((NC, N, DH), f32),
        mesh=mesh,
        compiler_params=sc_params,
        scratch_types=[
            pltpu.VMEM((NCH, CHUNK), jnp.int32),
            pltpu.VMEM((NCH, CHUNK), jnp.int32),
            pltpu.VMEM((NBUF, CHUNK, DH), f32),
            pltpu.VMEM_SHARED((N, DH), f32),
            pltpu.VMEM_SHARED((N, DH), f32),
            pltpu.SemaphoreType.DMA((NBUF,)),
            pltpu.SemaphoreType.DMA((NBUF,)),
        ],
    )
    def sc_seg(src_h, dst_h, tab_h, z_h, out_h,
               sidx_v, didx_v, rows_r, tab_sh, acc_sh, semg, sems):
        cid = lax.axis_index("c")
        sid = lax.axis_index("s")
        rb = (cid * NS + sid) * NCH
        sp = sid * STR
        pltpu.sync_copy(z_h.at[pl.ds(sp, STR)], acc_sh.at[pl.ds(sp, STR)])
        pltpu.sync_copy(tab_h.at[pl.ds(sp, STR)], tab_sh.at[pl.ds(sp, STR)])
        pltpu.sync_copy(src_h.at[pl.ds(rb, NCH)], sidx_v)
        pltpu.sync_copy(dst_h.at[pl.ds(rb, NCH)], didx_v)
        plsc.subcore_barrier()

        for b in range(NBUF):
            pltpu.async_copy(tab_sh.at[sidx_v.at[b]], rows_r.at[b], semg.at[b])

        @pl.loop(0, NCH + NBUF - (NCH % NBUF), step=NBUF)
        def _(k):
            for b in range(NBUF):
                c = k + b

                @pl.when(c < NCH)
                def _():
                    pltpu.make_async_copy(
                        tab_sh.at[sidx_v.at[0]], rows_r.at[b], semg.at[b]
                    ).wait()
                    pltpu.async_copy(
                        rows_r.at[b], acc_sh.at[didx_v.at[c]], sems.at[b],
                        add=True,
                    )

            for b in range(NBUF):
                c = k + b

                @pl.when(c < NCH)
                def _():
                    pltpu.make_async_copy(
                        rows_r.at[b], acc_sh.at[didx_v.at[0]], sems.at[b]
                    ).wait()

                @pl.when(c + NBUF < NCH)
                def _():
                    pltpu.async_copy(
                        tab_sh.at[sidx_v.at[c + NBUF]], rows_r.at[b],
                        semg.at[b],
                    )

        plsc.subcore_barrier()
        pltpu.sync_copy(acc_sh.at[pl.ds(sp, STR)], out_h.at[cid, pl.ds(sp, STR)])

    @functools.partial(
        pl.kernel,
        out_type=jax.ShapeDtypeStruct((E,), f32),
        mesh=mesh,
        compiler_params=sc_params_nl,
        scratch_types=[
            pltpu.VMEM((NCH, CHUNK), jnp.int32),
            pltpu.VMEM((NCH, CHUNK), jnp.int32),
            pltpu.VMEM((NSET, CHUNK, DO), f32),
            pltpu.VMEM((NSET, CHUNK, DO), f32),
            pltpu.VMEM((NSET * (CHUNK + 16),), f32),
            pltpu.VMEM_SHARED((N, DO), f32),
            pltpu.SemaphoreType.DMA((NSET,)),
            pltpu.SemaphoreType.DMA((NSET,)),
            pltpu.SemaphoreType.DMA((NSET,)),
        ],
    )
    def sc_dec(u_h, v_h, z_h, out_h,
               uix_v, vix_v, zu_r, zv_r, p_r, z_sh, semu, semv, semp):
        cid = lax.axis_index("c")
        sid = lax.axis_index("s")
        g = cid * NS + sid
        rb = g * NCH
        base = g * EPT
        sp = sid * STR
        lane15 = jax.lax.iota(jnp.int32, 16) == 15
        pltpu.sync_copy(z_h.at[pl.ds(sp, STR)], z_sh.at[pl.ds(sp, STR)])
        pltpu.sync_copy(u_h.at[pl.ds(rb, NCH)], uix_v)
        pltpu.sync_copy(v_h.at[pl.ds(rb, NCH)], vix_v)
        plsc.subcore_barrier()

        for s in range(NSET):
            pltpu.async_copy(z_sh.at[uix_v.at[s]], zu_r.at[s], semu.at[s])
            pltpu.async_copy(z_sh.at[vix_v.at[s]], zv_r.at[s], semv.at[s])

        @pl.loop(0, NCH + NSET - (NCH % NSET), step=NSET)
        def _(k):
            for s in range(NSET):
                c = k + s

                @pl.when(c < NCH)
                def _():
                    pltpu.make_async_copy(
                        z_sh.at[uix_v.at[0]], zu_r.at[s], semu.at[s]
                    ).wait()
                    pltpu.make_async_copy(
                        z_sh.at[vix_v.at[0]], zv_r.at[s], semv.at[s]
                    ).wait()

                    @pl.when(c >= NSET)
                    def _():
                        pltpu.make_async_copy(
                            p_r.at[pl.ds(s * (CHUNK + 16), CHUNK)],
                            out_h.at[pl.ds(0, CHUNK)], semp.at[s]
                        ).wait()

                    @pl.loop(0, CHUNK, step=8)
                    def _(r):
                        prods = []
                        for j in range(8):
                            a0 = zu_r[s, r + j, pl.ds(0, 16)]
                            a1 = zu_r[s, r + j, pl.ds(16, 16)]
                            b0 = zv_r[s, r + j, pl.ds(0, 16)]
                            b1 = zv_r[s, r + j, pl.ds(16, 16)]
                            prods.append(a0 * b0 + a1 * b1)
                        cums = [plsc.cumsum(p) for p in prods]
                        for j in range(8):
                            plsc.store_compressed(
                                p_r.at[pl.ds(s * (CHUNK + 16) + r + j, 16)],
                                cums[j], mask=lane15,
                            )

                    pltpu.async_copy(
                        p_r.at[pl.ds(s * (CHUNK + 16), CHUNK)],
                        out_h.at[pl.ds(base + c * CHUNK, CHUNK)],
                        semp.at[s],
                    )

                    @pl.when(c + NSET < NCH)
                    def _():
                        pltpu.async_copy(
                            z_sh.at[uix_v.at[c + NSET]], zu_r.at[s], semu.at[s]
                        )
                        pltpu.async_copy(
                            z_sh.at[vix_v.at[c + NSET]], zv_r.at[s], semv.at[s]
                        )

        for s in range(NSET):
            pltpu.make_async_copy(
                p_r.at[pl.ds(s * (CHUNK + 16), CHUNK)],
                out_h.at[pl.ds(0, CHUNK)], semp.at[s]
            ).wait()


    MB = N // 8

    def mm1_body(x_r, w_r, o_r):
        o_r[...] = jnp.dot(x_r[...], w_r[...], preferred_element_type=f32)

    def norm_body(d_r, h_r, g_r, s_r):
        deg = d_r[0, :, 0:1] + d_r[1, :, 0:1] + 1.0
        dis = lax.rsqrt(deg)
        s_r[...] = dis
        g_r[...] = dis * h_r[...]

    def layer2_body(a_r, h_r, s_r, w_r, b_r, g_r, hw_r):
        dis = s_r[...]
        acc = a_r[0] + a_r[1]
        h = jnp.maximum(dis * acc + (dis * dis) * h_r[...] + b_r[...], 0.0)
        hw2 = jnp.dot(h, w_r[...], preferred_element_type=f32)
        hw_r[...] = hw2
        g_r[...] = dis * hw2

    def zout_body(a_r, h_r, s_r, b_r, z_r):
        dis = s_r[...]
        acc = a_r[0] + a_r[1]
        z_r[...] = dis * acc + (dis * dis) * h_r[...] + b_r[...]


    degp = sc_deg(dst2, zeros16, e0)

    hW1 = pl.pallas_call(
        mm1_body,
        grid=(N // MB,),
        in_specs=[
            pl.BlockSpec((MB, DIN), lambda i: (i, 0)),
            pl.BlockSpec((DIN, DH), lambda i: (0, 0)),
        ],
        out_specs=pl.BlockSpec((MB, DH), lambda i: (i, 0)),
        out_shape=jax.ShapeDtypeStruct((N, DH), f32),
    )(x, W1)

    g1, dis = pl.pallas_call(
        norm_body,
        grid=(N // MB,),
        in_specs=[
            pl.BlockSpec((NC, MB, 16), lambda i: (0, i, 0)),
            pl.BlockSpec((MB, DH), lambda i: (i, 0)),
        ],
        out_specs=[
            pl.BlockSpec((MB, DH), lambda i: (i, 0)),
            pl.BlockSpec((MB, 1), lambda i: (i, 0)),
        ],
        out_shape=[
            jax.ShapeDtypeStruct((N, DH), f32),
            jax.ShapeDtypeStruct((N, 1), f32),
        ],
    )(degp, hW1)

    acc1 = sc_seg(src2, dst2, g1, zeros32)

    g2, hW2 = pl.pallas_call(
        layer2_body,
        grid=(N // MB,),
        in_specs=[
            pl.BlockSpec((NC, MB, DH), lambda i: (0, i, 0)),
            pl.BlockSpec((MB, DH), lambda i: (i, 0)),
            pl.BlockSpec((MB, 1), lambda i: (i, 0)),
            pl.BlockSpec((DH, DO), lambda i: (0, 0)),
            pl.BlockSpec((1, DH), lambda i: (0, 0)),
        ],
        out_specs=[
            pl.BlockSpec((MB, DO), lambda i: (i, 0)),
            pl.BlockSpec((MB, DO), lambda i: (i, 0)),
        ],
        out_shape=[
            jax.ShapeDtypeStruct((N, DO), f32),
            jax.ShapeDtypeStruct((N, DO), f32),
        ],
    )(acc1, hW1, dis, W2, b1.reshape(1, DH))

    acc2 = sc_seg(src2, dst2, g2, zeros32)

    z = pl.pallas_call(
        zout_body,
        grid=(N // MB,),
        in_specs=[
            pl.BlockSpec((NC, MB, DO), lambda i: (0, i, 0)),
            pl.BlockSpec((MB, DO), lambda i: (i, 0)),
            pl.BlockSpec((MB, 1), lambda i: (i, 0)),
            pl.BlockSpec((1, DO), lambda i: (0, 0)),
        ],
        out_specs=pl.BlockSpec((MB, DO), lambda i: (i, 0)),
        out_shape=jax.ShapeDtypeStruct((N, DO), f32),
    )(acc2, hW2, dis, b2.reshape(1, DO))

    return sc_dec(uu2, vv2, z)


# speedup vs baseline: 43.7676x; 1.0319x over previous
import functools

import jax
import jax.numpy as jnp
from jax import lax
from jax.experimental import pallas as pl
from jax.experimental.pallas import tpu as pltpu
from jax.experimental.pallas import tpu_sc as plsc

NC = 2
NS = 16
NW = NC * NS
CHUNK = 80
NBUF = 10
NSET = 3


def kernel(x, edge_index, edge_pairs, W1, b1, W2, b2):
    f32 = jnp.float32
    N0, DIN = x.shape
    DH = W1.shape[1]
    DO = W2.shape[1]
    E = edge_index.shape[1]

    N = ((N0 + 1023) // 1024) * 1024
    x = jnp.pad(x, ((0, N - N0), (0, 0)))

    EPT = E // NW
    NCH = EPT // CHUNK
    STR = N // NS

    src2 = edge_index[0].astype(jnp.int32).reshape(E // CHUNK, CHUNK)
    dst2 = edge_index[1].astype(jnp.int32).reshape(E // CHUNK, CHUNK)
    uu2 = edge_pairs[0].astype(jnp.int32).reshape(E // CHUNK, CHUNK)
    vv2 = edge_pairs[1].astype(jnp.int32).reshape(E // CHUNK, CHUNK)

    zeros16 = jnp.zeros((N, 16), f32)
    zeros32 = jnp.zeros((N, DH), f32)
    e0 = jnp.zeros((CHUNK, 16), f32).at[:, 0].set(1.0)

    mesh = plsc.VectorSubcoreMesh(core_axis_name="c", subcore_axis_name="s")
    sc_params = pltpu.CompilerParams(use_tc_tiling_on_sc=False)
    sc_params_nl = pltpu.CompilerParams(
        use_tc_tiling_on_sc=False, needs_layout_passes=False
    )


    @functools.partial(
        pl.kernel,
        out_type=jax.ShapeDtypeStruct((NC, N, 16), f32),
        mesh=mesh,
        compiler_params=sc_params,
        scratch_types=[
            pltpu.VMEM((NCH, CHUNK), jnp.int32),
            pltpu.VMEM((CHUNK, 16), f32),
            pltpu.VMEM_SHARED((N, 16), f32),
            pltpu.SemaphoreType.DMA,
        ],
    )
    def sc_deg(dst_h, z_h, e0_h, out_h, didx_v, e0_v, acc_sh, sem):
        cid = lax.axis_index("c")
        sid = lax.axis_index("s")
        rb = (cid * NS + sid) * NCH
        sp = sid * STR
        pltpu.sync_copy(z_h.at[pl.ds(sp, STR)], acc_sh.at[pl.ds(sp, STR)])
        pltpu.sync_copy(dst_h.at[pl.ds(rb, NCH)], didx_v)
        pltpu.sync_copy(e0_h, e0_v)
        plsc.subcore_barrier()

        @pl.loop(0, NCH)
        def _(k):
            pltpu.async_copy(e0_v, acc_sh.at[didx_v.at[k]], sem, add=True)

        @pl.loop(0, NCH)
        def _(k):
            pltpu.make_async_copy(e0_v, acc_sh.at[didx_v.at[0]], sem).wait()

        plsc.subcore_barrier()
        pltpu.sync_copy(acc_sh.at[pl.ds(sp, STR)], out_h.at[cid, pl.ds(sp, STR)])

    @functools.partial(
        pl.kernel,
        out_type=jax.ShapeDtypeStruct((NC, N, DH), f32),
        mesh=mesh,
        compiler_params=sc_params,
        scratch_types=[
            pltpu.VMEM((NCH, CHUNK), jnp.int32),
            pltpu.VMEM((NCH, CHUNK), jnp.int32),
            pltpu.VMEM((NBUF, CHUNK, DH), f32),
            pltpu.VMEM_SHARED((N, DH), f32),
            pltpu.VMEM_SHARED((N, DH), f32),
            pltpu.SemaphoreType.DMA((NBUF,)),
            pltpu.SemaphoreType.DMA((NBUF,)),
        ],
    )
    def sc_seg(src_h, dst_h, tab_h, z_h, out_h,
               sidx_v, didx_v, rows_r, tab_sh, acc_sh, semg, sems):
        cid = lax.axis_index("c")
        sid = lax.axis_index("s")
        rb = (cid * NS + sid) * NCH
        sp = sid * STR
        pltpu.sync_copy(z_h.at[pl.ds(sp, STR)], acc_sh.at[pl.ds(sp, STR)])
        pltpu.sync_copy(tab_h.at[pl.ds(sp, STR)], tab_sh.at[pl.ds(sp, STR)])
        pltpu.sync_copy(src_h.at[pl.ds(rb, NCH)], sidx_v)
        pltpu.sync_copy(dst_h.at[pl.ds(rb, NCH)], didx_v)
        plsc.subcore_barrier()

        for b in range(NBUF):
            pltpu.async_copy(tab_sh.at[sidx_v.at[b]], rows_r.at[b], semg.at[b])

        @pl.loop(0, NCH + NBUF - (NCH % NBUF), step=NBUF)
        def _(k):
            for b in range(NBUF):
                c = k + b

                @pl.when(c < NCH)
                def _():
                    pltpu.make_async_copy(
                        tab_sh.at[sidx_v.at[0]], rows_r.at[b], semg.at[b]
                    ).wait()
                    pltpu.async_copy(
                        rows_r.at[b], acc_sh.at[didx_v.at[c]], sems.at[b],
                        add=True,
                    )

            for b in range(NBUF):
                c = k + b

                @pl.when(c < NCH)
                def _():
                    pltpu.make_async_copy(
                        rows_r.at[b], acc_sh.at[didx_v.at[0]], sems.at[b]
                    ).wait()

                @pl.when(c + NBUF < NCH)
                def _():
                    pltpu.async_copy(
                        tab_sh.at[sidx_v.at[c + NBUF]], rows_r.at[b],
                        semg.at[b],
                    )

        plsc.subcore_barrier()
        pltpu.sync_copy(acc_sh.at[pl.ds(sp, STR)], out_h.at[cid, pl.ds(sp, STR)])

    @functools.partial(
        pl.kernel,
        out_type=jax.ShapeDtypeStruct((E,), f32),
        mesh=mesh,
        compiler_params=sc_params_nl,
        scratch_types=[
            pltpu.VMEM((NCH, CHUNK), jnp.int32),
            pltpu.VMEM((NCH, CHUNK), jnp.int32),
            pltpu.VMEM((NSET, CHUNK, DO), f32),
            pltpu.VMEM((NSET, CHUNK, DO), f32),
            pltpu.VMEM((NSET * (CHUNK + 16),), f32),
            pltpu.VMEM_SHARED((N, DO), f32),
            pltpu.SemaphoreType.DMA((NSET,)),
            pltpu.SemaphoreType.DMA((NSET,)),
            pltpu.SemaphoreType.DMA((NSET,)),
        ],
    )
    def sc_dec(u_h, v_h, z_h, out_h,
               uix_v, vix_v, zu_r, zv_r, p_r, z_sh, semu, semv, semp):
        cid = lax.axis_index("c")
        sid = lax.axis_index("s")
        g = cid * NS + sid
        rb = g * NCH
        base = g * EPT
        sp = sid * STR
        lane15 = jax.lax.iota(jnp.int32, 16) == 15
        pltpu.sync_copy(z_h.at[pl.ds(sp, STR)], z_sh.at[pl.ds(sp, STR)])
        pltpu.sync_copy(u_h.at[pl.ds(rb, NCH)], uix_v)
        pltpu.sync_copy(v_h.at[pl.ds(rb, NCH)], vix_v)
        plsc.subcore_barrier()

        for s in range(NSET):
            pltpu.async_copy(z_sh.at[uix_v.at[s]], zu_r.at[s], semu.at[s])
            pltpu.async_copy(z_sh.at[vix_v.at[s]], zv_r.at[s], semv.at[s])

        @pl.loop(0, NCH + NSET - (NCH % NSET), step=NSET)
        def _(k):
            for s in range(NSET):
                c = k + s

                @pl.when(c < NCH)
                def _():
                    pltpu.make_async_copy(
                        z_sh.at[uix_v.at[0]], zu_r.at[s], semu.at[s]
                    ).wait()
                    pltpu.make_async_copy(
                        z_sh.at[vix_v.at[0]], zv_r.at[s], semv.at[s]
                    ).wait()

                    @pl.when(c >= NSET)
                    def _():
                        pltpu.make_async_copy(
                            p_r.at[pl.ds(s * (CHUNK + 16), CHUNK)],
                            out_h.at[pl.ds(0, CHUNK)], semp.at[s]
                        ).wait()

                    @pl.loop(0, CHUNK, step=16)
                    def _(r):
                        prods = []
                        for j in range(16):
                            a0 = zu_r[s, r + j, pl.ds(0, 16)]
                            a1 = zu_r[s, r + j, pl.ds(16, 16)]
                            b0 = zv_r[s, r + j, pl.ds(0, 16)]
                            b1 = zv_r[s, r + j, pl.ds(16, 16)]
                            prods.append(a0 * b0 + a1 * b1)
                        cums = [plsc.cumsum(p) for p in prods]
                        for j in range(16):
                            plsc.store_compressed(
                                p_r.at[pl.ds(s * (CHUNK + 16) + r + j, 16)],
                                cums[j], mask=lane15,
                            )

                    pltpu.async_copy(
                        p_r.at[pl.ds(s * (CHUNK + 16), CHUNK)],
                        out_h.at[pl.ds(base + c * CHUNK, CHUNK)],
                        semp.at[s],
                    )

                    @pl.when(c + NSET < NCH)
                    def _():
                        pltpu.async_copy(
                            z_sh.at[uix_v.at[c + NSET]], zu_r.at[s], semu.at[s]
                        )
                        pltpu.async_copy(
                            z_sh.at[vix_v.at[c + NSET]], zv_r.at[s], semv.at[s]
                        )

        for s in range(NSET):
            pltpu.make_async_copy(
                p_r.at[pl.ds(s * (CHUNK + 16), CHUNK)],
                out_h.at[pl.ds(0, CHUNK)], semp.at[s]
            ).wait()


    MB = N // 8

    def mm1_body(x_r, w_r, o_r):
        o_r[...] = jnp.dot(x_r[...], w_r[...], preferred_element_type=f32)

    def norm_body(d_r, h_r, g_r, s_r):
        deg = d_r[0, :, 0:1] + d_r[1, :, 0:1] + 1.0
        dis = lax.rsqrt(deg)
        s_r[...] = dis
        g_r[...] = dis * h_r[...]

    def layer2_body(a_r, h_r, s_r, w_r, b_r, g_r, hw_r):
        dis = s_r[...]
        acc = a_r[0] + a_r[1]
        h = jnp.maximum(dis * acc + (dis * dis) * h_r[...] + b_r[...], 0.0)
        hw2 = jnp.dot(h, w_r[...], preferred_element_type=f32)
        hw_r[...] = hw2
        g_r[...] = dis * hw2

    def zout_body(a_r, h_r, s_r, b_r, z_r):
        dis = s_r[...]
        acc = a_r[0] + a_r[1]
        z_r[...] = dis * acc + (dis * dis) * h_r[...] + b_r[...]


    degp = sc_deg(dst2, zeros16, e0)

    hW1 = pl.pallas_call(
        mm1_body,
        grid=(N // MB,),
        in_specs=[
            pl.BlockSpec((MB, DIN), lambda i: (i, 0)),
            pl.BlockSpec((DIN, DH), lambda i: (0, 0)),
        ],
        out_specs=pl.BlockSpec((MB, DH), lambda i: (i, 0)),
        out_shape=jax.ShapeDtypeStruct((N, DH), f32),
    )(x, W1)

    g1, dis = pl.pallas_call(
        norm_body,
        grid=(N // MB,),
        in_specs=[
            pl.BlockSpec((NC, MB, 16), lambda i: (0, i, 0)),
            pl.BlockSpec((MB, DH), lambda i: (i, 0)),
        ],
        out_specs=[
            pl.BlockSpec((MB, DH), lambda i: (i, 0)),
            pl.BlockSpec((MB, 1), lambda i: (i, 0)),
        ],
        out_shape=[
            jax.ShapeDtypeStruct((N, DH), f32),
            jax.ShapeDtypeStruct((N, 1), f32),
        ],
    )(degp, hW1)

    acc1 = sc_seg(src2, dst2, g1, zeros32)

    g2, hW2 = pl.pallas_call(
        layer2_body,
        grid=(N // MB,),
        in_specs=[
            pl.BlockSpec((NC, MB, DH), lambda i: (0, i, 0)),
            pl.BlockSpec((MB, DH), lambda i: (i, 0)),
            pl.BlockSpec((MB, 1), lambda i: (i, 0)),
            pl.BlockSpec((DH, DO), lambda i: (0, 0)),
            pl.BlockSpec((1, DH), lambda i: (0, 0)),
        ],
        out_specs=[
            pl.BlockSpec((MB, DO), lambda i: (i, 0)),
            pl.BlockSpec((MB, DO), lambda i: (i, 0)),
        ],
        out_shape=[
            jax.ShapeDtypeStruct((N, DO), f32),
            jax.ShapeDtypeStruct((N, DO), f32),
        ],
    )(acc1, hW1, dis, W2, b1.reshape(1, DH))

    acc2 = sc_seg(src2, dst2, g2, zeros32)

    z = pl.pallas_call(
        zout_body,
        grid=(N // MB,),
        in_specs=[
            pl.BlockSpec((NC, MB, DO), lambda i: (0, i, 0)),
            pl.BlockSpec((MB, DO), lambda i: (i, 0)),
            pl.BlockSpec((MB, 1), lambda i: (i, 0)),
            pl.BlockSpec((1, DO), lambda i: (0, 0)),
        ],
        out_specs=pl.BlockSpec((MB, DO), lambda i: (i, 0)),
        out_shape=jax.ShapeDtypeStruct((N, DO), f32),
    )(acc2, hW2, dis, b2.reshape(1, DO))

    return sc_dec(uu2, vv2, z)
